# Initial kernel scaffold; baseline (speedup 1.0000x reference)
import jax
import jax.numpy as jnp
from jax.experimental import pallas as pl


def kernel(V, E, X, W, b):
    raise NotImplementedError("write your pallas kernel here")


# speedup vs baseline: 12.8596x; 12.8596x over previous
import functools

import jax
import jax.numpy as jnp
from jax import lax
from jax.experimental import pallas as pl
from jax.experimental.pallas import tpu as pltpu
from jax.experimental.pallas import tpu_sc as plsc

N = 10000
D = 128
E_TOT = 320000
NC = 2
NS = 16
NW = NC * NS
EPT = E_TOT // NW
CH = 80
NCHUNK = EPT // CH
NP = 10240
RPT = NP // NS
HL = 128

_mesh = plsc.VectorSubcoreMesh(core_axis_name="c", subcore_axis_name="s")


@functools.partial(
    pl.kernel,
    mesh=_mesh,
    out_type=jax.ShapeDtypeStruct((NC * NP, HL), jnp.float32),
    scratch_types=[
        pltpu.VMEM((NCHUNK, CH), jnp.int32),
        pltpu.VMEM((CH, HL), jnp.float32),
        pltpu.VMEM_SHARED((NP, HL), jnp.float32),
    ],
)
def _sc_hist(dst_hbm, ones_hbm, zrow_hbm, out_hbm, dstv, onesv, hist):
    c = lax.axis_index("c")
    s = lax.axis_index("s")
    wid = c * NS + s
    r0 = s * RPT
    pltpu.sync_copy(zrow_hbm, hist.at[pl.ds(r0, RPT)])
    pltpu.sync_copy(ones_hbm, onesv)
    pltpu.sync_copy(dst_hbm.at[wid], dstv)
    plsc.subcore_barrier()

    def chunk(i, carry):
        pltpu.sync_copy(onesv, hist.at[dstv.at[i]], add=True)
        return carry

    lax.fori_loop(0, NCHUNK, chunk, 0)
    plsc.subcore_barrier()
    pltpu.sync_copy(hist.at[pl.ds(r0, RPT)],
                    out_hbm.at[pl.ds(c * NP + r0, RPT)])


@functools.partial(
    pl.kernel,
    mesh=_mesh,
    out_type=jax.ShapeDtypeStruct((NC * NP, D), jnp.float32),
    scratch_types=[
        pltpu.VMEM((NCHUNK, CH), jnp.int32),
        pltpu.VMEM((NCHUNK, CH), jnp.int32),
        pltpu.VMEM((CH, D), jnp.float32),
        pltpu.VMEM_SHARED((NP, D), jnp.float32),
        pltpu.SemaphoreType.DMA,
    ],
)
def _sc_round(src_hbm, dst_hbm, g_hbm, zrows_hbm, out_hbm,
              srcv, dstv, rows, agg, sem):
    c = lax.axis_index("c")
    s = lax.axis_index("s")
    wid = c * NS + s
    r0 = s * RPT
    @pl.when(c == 0)
    def _():
        pltpu.sync_copy(g_hbm.at[pl.ds(r0, RPT)], agg.at[pl.ds(r0, RPT)])

    @pl.when(c != 0)
    def _():
        pltpu.sync_copy(zrows_hbm, agg.at[pl.ds(r0, RPT)])

    pltpu.sync_copy(src_hbm.at[wid], srcv)
    pltpu.sync_copy(dst_hbm.at[wid], dstv)
    plsc.subcore_barrier()

    def chunk(i, carry):
        pltpu.async_copy(g_hbm.at[srcv.at[i]], rows, sem).wait()
        pltpu.sync_copy(rows, agg.at[dstv.at[i]], add=True)
        return carry

    lax.fori_loop(0, NCHUNK, chunk, 0)
    plsc.subcore_barrier()
    pltpu.sync_copy(agg.at[pl.ds(r0, RPT)],
                    out_hbm.at[pl.ds(c * NP + r0, RPT)])


def _tc_prep_body(h0_ref, h1_ref, x_ref, dinv_ref, g1_ref):
    deg = h0_ref[:, 0:1] + h1_ref[:, 0:1] + 1.0
    dinv = lax.rsqrt(deg)
    dinv_ref[...] = dinv
    g1_ref[...] = dinv * x_ref[...]


def _tc_comb_body(p0_ref, p1_ref, dinv_ref, g2_ref):
    dv = dinv_ref[...]
    g2_ref[...] = (dv * dv) * (p0_ref[...] + p1_ref[...])


def _tc_final_body(p0_ref, p1_ref, dinv_ref, w_ref, b_ref, out_ref):
    h = dinv_ref[...] * (p0_ref[...] + p1_ref[...])
    out_ref[...] = (
        jnp.dot(h, w_ref[...], preferred_element_type=jnp.float32) + b_ref[...]
    )


_BR = 2000


def _tc_prep(h0, h1, x):
    return pl.pallas_call(
        _tc_prep_body,
        grid=(N // _BR,),
        in_specs=[
            pl.BlockSpec((_BR, HL), lambda i: (i, 0)),
            pl.BlockSpec((_BR, HL), lambda i: (i, 0)),
            pl.BlockSpec((_BR, D), lambda i: (i, 0)),
        ],
        out_specs=[
            pl.BlockSpec((_BR, 1), lambda i: (i, 0)),
            pl.BlockSpec((_BR, D), lambda i: (i, 0)),
        ],
        out_shape=[
            jax.ShapeDtypeStruct((N, 1), jnp.float32),
            jax.ShapeDtypeStruct((N, D), jnp.float32),
        ],
    )(h0, h1, x)


def _tc_comb(p0, p1, dinv):
    return pl.pallas_call(
        _tc_comb_body,
        grid=(N // _BR,),
        in_specs=[
            pl.BlockSpec((_BR, D), lambda i: (i, 0)),
            pl.BlockSpec((_BR, D), lambda i: (i, 0)),
            pl.BlockSpec((_BR, 1), lambda i: (i, 0)),
        ],
        out_specs=pl.BlockSpec((_BR, D), lambda i: (i, 0)),
        out_shape=jax.ShapeDtypeStruct((N, D), jnp.float32),
    )(p0, p1, dinv)


def _tc_final(p0, p1, dinv, w, b2):
    return pl.pallas_call(
        _tc_final_body,
        grid=(N // _BR,),
        in_specs=[
            pl.BlockSpec((_BR, D), lambda i: (i, 0)),
            pl.BlockSpec((_BR, D), lambda i: (i, 0)),
            pl.BlockSpec((_BR, 1), lambda i: (i, 0)),
            pl.BlockSpec((D, D), lambda i: (0, 0)),
            pl.BlockSpec((1, D), lambda i: (0, 0)),
        ],
        out_specs=pl.BlockSpec((_BR, D), lambda i: (i, 0)),
        out_shape=jax.ShapeDtypeStruct((N, D), jnp.float32),
    )(p0, p1, dinv, w, b2)


def _pad_rows(a):
    return jnp.concatenate([a, jnp.zeros((NP - N,) + a.shape[1:], a.dtype)], axis=0)


def kernel(V, E, X, W, b):
    src = E[0].astype(jnp.int32).reshape(NW, NCHUNK, CH)
    dst = E[1].astype(jnp.int32).reshape(NW, NCHUNK, CH)
    ones_rows = jnp.ones((CH, HL), jnp.float32)
    zrow_h = jnp.zeros((RPT, HL), jnp.float32)
    zrows = jnp.zeros((RPT, D), jnp.float32)

    hist = _sc_hist(dst, ones_rows, zrow_h)
    dinv, g1 = _tc_prep(hist[:N], hist[NP:NP + N], X)
    p1 = _sc_round(src, dst, _pad_rows(g1), zrows)
    g2 = _tc_comb(p1[:N], p1[NP:NP + N], dinv)
    p2 = _sc_round(src, dst, _pad_rows(g2), zrows)
    return _tc_final(p2[:N], p2[NP:NP + N], dinv, W, b.reshape(1, D))


# speedup vs baseline: 12.9975x; 1.0107x over previous
import functools

import jax
import jax.numpy as jnp
from jax import lax
from jax.experimental import pallas as pl
from jax.experimental.pallas import tpu as pltpu
from jax.experimental.pallas import tpu_sc as plsc

N = 10000
D = 128
E_TOT = 320000
NC = 2
NS = 16
NW = NC * NS
EPT = E_TOT // NW
CH = 80
NCHUNK = EPT // CH
NP = 10240
RPT = NP // NS
HR = NP // D

_mesh = plsc.VectorSubcoreMesh(core_axis_name="c", subcore_axis_name="s")


@functools.partial(
    pl.kernel,
    mesh=_mesh,
    out_type=jax.ShapeDtypeStruct((NC * NP, D), jnp.float32),
    scratch_types=[
        pltpu.VMEM((NCHUNK, CH), jnp.int32),
        pltpu.VMEM((CH, D), jnp.float32),
        pltpu.VMEM_SHARED((NP, D), jnp.float32),
    ],
)
def _sc_hist(dst_hbm, ones_hbm, zrows_hbm, out_hbm, dstv, ones, hist):
    c = lax.axis_index("c")
    s = lax.axis_index("s")
    wid = c * NS + s
    r0 = s * RPT
    pltpu.sync_copy(zrows_hbm, hist.at[pl.ds(r0, RPT)])
    pltpu.sync_copy(dst_hbm.at[wid], dstv)
    pltpu.sync_copy(ones_hbm, ones)
    plsc.subcore_barrier()

    def step(i, carry):
        pltpu.sync_copy(ones, hist.at[dstv.at[i]], add=True)
        return carry

    lax.fori_loop(0, NCHUNK, step, 0)
    plsc.subcore_barrier()
    pltpu.sync_copy(hist.at[pl.ds(r0, RPT)],
                    out_hbm.at[pl.ds(c * NP + r0, RPT)])


@functools.partial(
    pl.kernel,
    mesh=_mesh,
    out_type=jax.ShapeDtypeStruct((NC * NP, D), jnp.float32),
    scratch_types=[
        pltpu.VMEM((2, NCHUNK, CH), jnp.int32),
        pltpu.VMEM((CH, D), jnp.float32),
        pltpu.VMEM_SHARED((NP, D), jnp.float32),
        pltpu.SemaphoreType.DMA,
    ],
)
def _sc_round(src_hbm, dst_hbm, g_hbm, zrows_hbm, out_hbm,
              idxv, rows, agg, sem0):
    srcv = idxv.at[0]
    dstv = idxv.at[1]
    c = lax.axis_index("c")
    s = lax.axis_index("s")
    wid = c * NS + s
    r0 = s * RPT
    @pl.when(c == 0)
    def _():
        pltpu.sync_copy(g_hbm.at[pl.ds(r0, RPT)], agg.at[pl.ds(r0, RPT)])

    @pl.when(c != 0)
    def _():
        pltpu.sync_copy(zrows_hbm, agg.at[pl.ds(r0, RPT)])

    pltpu.sync_copy(src_hbm.at[wid], srcv)
    pltpu.sync_copy(dst_hbm.at[wid], dstv)
    plsc.subcore_barrier()

    pltpu.async_copy(g_hbm.at[srcv.at[0]], rows, sem0)

    def step(k, carry):
        pltpu.make_async_copy(g_hbm.at[srcv.at[k]], rows, sem0).wait()
        pltpu.sync_copy(rows, agg.at[dstv.at[k]], add=True)

        @pl.when(k + 1 < NCHUNK)
        def _():
            pltpu.async_copy(g_hbm.at[srcv.at[k + 1]], rows, sem0)
        return carry

    lax.fori_loop(0, NCHUNK, step, 0)
    plsc.subcore_barrier()
    pltpu.sync_copy(agg.at[pl.ds(r0, RPT)],
                    out_hbm.at[pl.ds(c * NP + r0, RPT)])


def _tc_prep_body(d0_ref, d1_ref, x_ref, dinv_ref, g1_ref):
    deg = d0_ref[:, 0:1] + d1_ref[:, 0:1] + 1.0
    dinv = lax.rsqrt(deg)
    dinv_ref[...] = dinv
    g1_ref[...] = dinv * x_ref[...]


def _tc_comb_body(p0_ref, p1_ref, dinv_ref, g2_ref):
    dv = dinv_ref[...]
    g2_ref[...] = (dv * dv) * (p0_ref[...] + p1_ref[...])


def _tc_final_body(p0_ref, p1_ref, dinv_ref, w_ref, b_ref, out_ref):
    h = dinv_ref[...] * (p0_ref[...] + p1_ref[...])
    out_ref[...] = (
        jnp.dot(h, w_ref[...], preferred_element_type=jnp.float32) + b_ref[...]
    )


_BR = 2048


def _tc_prep(d0, d1, x):
    return pl.pallas_call(
        _tc_prep_body,
        grid=(NP // _BR,),
        in_specs=[
            pl.BlockSpec((_BR, D), lambda i: (i, 0)),
            pl.BlockSpec((_BR, D), lambda i: (i, 0)),
            pl.BlockSpec((_BR, D), lambda i: (i, 0)),
        ],
        out_specs=[
            pl.BlockSpec((_BR, 1), lambda i: (i, 0)),
            pl.BlockSpec((_BR, D), lambda i: (i, 0)),
        ],
        out_shape=[
            jax.ShapeDtypeStruct((NP, 1), jnp.float32),
            jax.ShapeDtypeStruct((NP, D), jnp.float32),
        ],
    )(d0, d1, x)


def _tc_comb(p0, p1, dinv):
    return pl.pallas_call(
        _tc_comb_body,
        grid=(NP // _BR,),
        in_specs=[
            pl.BlockSpec((_BR, D), lambda i: (i, 0)),
            pl.BlockSpec((_BR, D), lambda i: (i, 0)),
            pl.BlockSpec((_BR, 1), lambda i: (i, 0)),
        ],
        out_specs=pl.BlockSpec((_BR, D), lambda i: (i, 0)),
        out_shape=jax.ShapeDtypeStruct((NP, D), jnp.float32),
    )(p0, p1, dinv)


def _tc_final(p0, p1, dinv, w, b2):
    return pl.pallas_call(
        _tc_final_body,
        grid=(NP // _BR,),
        in_specs=[
            pl.BlockSpec((_BR, D), lambda i: (i, 0)),
            pl.BlockSpec((_BR, D), lambda i: (i, 0)),
            pl.BlockSpec((_BR, 1), lambda i: (i, 0)),
            pl.BlockSpec((D, D), lambda i: (0, 0)),
            pl.BlockSpec((1, D), lambda i: (0, 0)),
        ],
        out_specs=pl.BlockSpec((_BR, D), lambda i: (i, 0)),
        out_shape=jax.ShapeDtypeStruct((N, D), jnp.float32),
    )(p0, p1, dinv, w, b2)


def kernel(V, E, X, W, b):
    src = E[0].astype(jnp.int32).reshape(NW, NCHUNK, CH)
    dst = E[1].astype(jnp.int32).reshape(NW, NCHUNK, CH)
    ones = jnp.ones((CH, D), jnp.float32)
    zrows = jnp.zeros((RPT, D), jnp.float32)

    hist = _sc_hist(dst, ones, zrows)
    d0 = hist[:NP]
    d1 = hist[NP:]
    dinv, g1 = _tc_prep(d0, d1, X)
    p1 = _sc_round(src, dst, g1, zrows)
    g2 = _tc_comb(p1[:NP], p1[NP:], dinv)
    p2 = _sc_round(src, dst, g2, zrows)
    return _tc_final(p2[:NP], p2[NP:], dinv, W, b.reshape(1, D))


# speedup vs baseline: 18.6192x; 1.4325x over previous
import functools

import jax
import jax.numpy as jnp
from jax import lax
from jax.experimental import pallas as pl
from jax.experimental.pallas import tpu as pltpu
from jax.experimental.pallas import tpu_sc as plsc

N = 10000
D = 128
E_TOT = 320000
NC = 2
NS = 16
NW = NC * NS
EPT = E_TOT // NW
CH = 80
NCHUNK = EPT // CH
NP = 10240
RPT = NP // NS
HR = NP // D

_mesh = plsc.VectorSubcoreMesh(core_axis_name="c", subcore_axis_name="s")


@functools.partial(
    pl.kernel,
    mesh=_mesh,
    out_type=jax.ShapeDtypeStruct((NC * NP, D), jnp.float32),
    scratch_types=[
        pltpu.VMEM((NCHUNK, CH), jnp.int32),
        pltpu.VMEM((CH, D), jnp.float32),
        pltpu.VMEM_SHARED((NP, D), jnp.float32),
    ],
)
def _sc_hist(dst_hbm, ones_hbm, zrows_hbm, out_hbm, dstv, ones, hist):
    c = lax.axis_index("c")
    s = lax.axis_index("s")
    wid = c * NS + s
    r0 = s * RPT
    pltpu.sync_copy(zrows_hbm, hist.at[pl.ds(r0, RPT)])
    pltpu.sync_copy(dst_hbm.at[wid], dstv)
    pltpu.sync_copy(ones_hbm, ones)
    plsc.subcore_barrier()

    def step(i, carry):
        pltpu.sync_copy(ones, hist.at[dstv.at[i]], add=True)
        return carry

    lax.fori_loop(0, NCHUNK, step, 0)
    plsc.subcore_barrier()
    pltpu.sync_copy(hist.at[pl.ds(r0, RPT)],
                    out_hbm.at[pl.ds(c * NP + r0, RPT)])


RQ = 16
NCP = 128
@functools.partial(
    pl.kernel,
    mesh=_mesh,
    out_type=jax.ShapeDtypeStruct((NC * NP, D), jnp.float32),
    scratch_types=[
        pltpu.VMEM((NCHUNK, CH), jnp.int32),
        pltpu.VMEM((RQ, CH), jnp.int32),
        pltpu.VMEM((2, CH, D), jnp.float32),
        pltpu.VMEM_SHARED((NP, D), jnp.float32),
        pltpu.SemaphoreType.DMA,
        pltpu.SemaphoreType.DMA,
        pltpu.SemaphoreType.DMA,
        pltpu.SemaphoreType.DMA,
    ],
)
def _sc_round(src_hbm, dst_hbm, g_hbm, zrows_hbm, out_hbm,
              srcv, dstr, rowsv, agg, g0, g1, r0s, r1s):
    gsem = (g0, g1)
    rsem = (r0s, r1s)
    c = lax.axis_index("c")
    s = lax.axis_index("s")
    wid = c * NS + s
    r0 = s * RPT
    @pl.when(c == 0)
    def _():
        pltpu.sync_copy(g_hbm.at[pl.ds(r0, RPT)], agg.at[pl.ds(r0, RPT)])

    @pl.when(c != 0)
    def _():
        pltpu.sync_copy(zrows_hbm, agg.at[pl.ds(r0, RPT)])

    pltpu.sync_copy(src_hbm.at[wid], srcv)
    pltpu.sync_copy(dst_hbm.at[wid].at[pl.ds(0, RQ)], dstr)
    plsc.subcore_barrier()

    for k in range(2):
        pltpu.async_copy(g_hbm.at[srcv.at[k]], rowsv.at[k], gsem[k])

    for k in range(NCHUNK):
        b = k % 2
        if k % 8 == 0 and k >= RQ:
            pltpu.make_async_copy(dst_hbm.at[wid].at[pl.ds(k, 8)],
                                  dstr.at[pl.ds(k % RQ, 8)],
                                  rsem[(k // 8) % 2]).wait()
        pltpu.make_async_copy(g_hbm.at[srcv.at[k]], rowsv.at[b],
                              gsem[b]).wait()
        pltpu.sync_copy(rowsv.at[b], agg.at[dstr.at[k % RQ]], add=True)
        if k % 8 == 7:
            base = k + 9
            if base <= NCP - 8:
                pltpu.async_copy(dst_hbm.at[wid].at[pl.ds(base, 8)],
                                 dstr.at[pl.ds(base % RQ, 8)],
                                 rsem[(base // 8) % 2])
        j = k + 2
        if j < NCHUNK:
            pltpu.async_copy(g_hbm.at[srcv.at[j]], rowsv.at[b], gsem[b])

    plsc.subcore_barrier()
    pltpu.sync_copy(agg.at[pl.ds(r0, RPT)],
                    out_hbm.at[pl.ds(c * NP + r0, RPT)])


def _tc_prep_body(d0_ref, d1_ref, x_ref, dinv_ref, g1_ref):
    deg = d0_ref[:, 0:1] + d1_ref[:, 0:1] + 1.0
    dinv = lax.rsqrt(deg)
    dinv_ref[...] = dinv
    g1_ref[...] = dinv * x_ref[...]


def _tc_comb_body(p0_ref, p1_ref, dinv_ref, g2_ref):
    dv = dinv_ref[...]
    g2_ref[...] = (dv * dv) * (p0_ref[...] + p1_ref[...])


def _tc_final_body(p0_ref, p1_ref, dinv_ref, w_ref, b_ref, out_ref):
    h = dinv_ref[...] * (p0_ref[...] + p1_ref[...])
    out_ref[...] = (
        jnp.dot(h, w_ref[...], preferred_element_type=jnp.float32) + b_ref[...]
    )


_BR = 2048


def _tc_prep(d0, d1, x):
    return pl.pallas_call(
        _tc_prep_body,
        grid=(NP // _BR,),
        in_specs=[
            pl.BlockSpec((_BR, D), lambda i: (i, 0)),
            pl.BlockSpec((_BR, D), lambda i: (i, 0)),
            pl.BlockSpec((_BR, D), lambda i: (i, 0)),
        ],
        out_specs=[
            pl.BlockSpec((_BR, 1), lambda i: (i, 0)),
            pl.BlockSpec((_BR, D), lambda i: (i, 0)),
        ],
        out_shape=[
            jax.ShapeDtypeStruct((NP, 1), jnp.float32),
            jax.ShapeDtypeStruct((NP, D), jnp.float32),
        ],
    )(d0, d1, x)


def _tc_comb(p0, p1, dinv):
    return pl.pallas_call(
        _tc_comb_body,
        grid=(NP // _BR,),
        in_specs=[
            pl.BlockSpec((_BR, D), lambda i: (i, 0)),
            pl.BlockSpec((_BR, D), lambda i: (i, 0)),
            pl.BlockSpec((_BR, 1), lambda i: (i, 0)),
        ],
        out_specs=pl.BlockSpec((_BR, D), lambda i: (i, 0)),
        out_shape=jax.ShapeDtypeStruct((NP, D), jnp.float32),
    )(p0, p1, dinv)


def _tc_final(p0, p1, dinv, w, b2):
    return pl.pallas_call(
        _tc_final_body,
        grid=(NP // _BR,),
        in_specs=[
            pl.BlockSpec((_BR, D), lambda i: (i, 0)),
            pl.BlockSpec((_BR, D), lambda i: (i, 0)),
            pl.BlockSpec((_BR, 1), lambda i: (i, 0)),
            pl.BlockSpec((D, D), lambda i: (0, 0)),
            pl.BlockSpec((1, D), lambda i: (0, 0)),
        ],
        out_specs=pl.BlockSpec((_BR, D), lambda i: (i, 0)),
        out_shape=jax.ShapeDtypeStruct((N, D), jnp.float32),
    )(p0, p1, dinv, w, b2)


def kernel(V, E, X, W, b):
    src = E[0].astype(jnp.int32).reshape(NW, NCHUNK, CH)
    dst = E[1].astype(jnp.int32).reshape(NW, NCHUNK, CH)
    dstp = jnp.pad(dst, ((0, 0), (0, NCP - NCHUNK), (0, 0)))
    ones = jnp.ones((CH, D), jnp.float32)
    zrows = jnp.zeros((RPT, D), jnp.float32)

    hist = _sc_hist(dst, ones, zrows)
    d0 = hist[:NP]
    d1 = hist[NP:]
    dinv, g1 = _tc_prep(d0, d1, X)
    p1 = _sc_round(src, dstp, g1, zrows)
    g2 = _tc_comb(p1[:NP], p1[NP:], dinv)
    p2 = _sc_round(src, dstp, g2, zrows)
    return _tc_final(p2[:NP], p2[NP:], dinv, W, b.reshape(1, D))
